# baseline (speedup 1.0000x reference)
import jax
import jax.numpy as jnp
from jax.experimental import pallas as pl

B = 16384
N = 49
TILE = 512


def _fused_net_kernel(x_ref, k1_ref, c1_ref, k2_ref, c2_ref, w31_ref, b31_ref,
                      w2_ref, b2_ref, wmu_ref, cmu_ref,
                      scale_ref, bias_ref, out_ref):
    x = x_ref[...]
    h = jax.nn.relu(jnp.dot(x, k1_ref[...], preferred_element_type=jnp.float32,
                            precision=jax.lax.Precision.HIGHEST)
                    + c1_ref[...])
    h = jax.nn.relu(jnp.dot(h, k2_ref[...], preferred_element_type=jnp.float32,
                            precision=jax.lax.Precision.HIGHEST)
                    + c2_ref[...])
    h = jax.nn.relu(jnp.dot(h, w31_ref[...], preferred_element_type=jnp.float32,
                            precision=jax.lax.Precision.HIGHEST)
                    + b31_ref[...])
    h = jax.nn.relu(jnp.dot(h, w2_ref[...], preferred_element_type=jnp.float32,
                            precision=jax.lax.Precision.HIGHEST)
                    + b2_ref[...])
    mu = jnp.tanh(jnp.dot(h, wmu_ref[...], preferred_element_type=jnp.float32,
                            precision=jax.lax.Precision.HIGHEST)
                  + cmu_ref[...])
    out_ref[...] = mu * scale_ref[...] + bias_ref[...]


def _fold_sage(Amean, Wl, bl, Wr):
    ci, co = Wl.shape
    eye = jnp.eye(N, dtype=jnp.float32)
    K = (Amean.T[:, None, :, None] * Wl[None, :, None, :]
         + eye[:, None, :, None] * Wr[None, :, None, :])
    K = K.reshape(N * ci, N * co)
    c = jnp.tile(bl, N)
    return K, c


def kernel(x, Wl1, bl1, Wr1, Wl2, bl2, Wr2, Wl3, bl3, Wr3, W1, b1, W2, b2,
           Wmu, bmu, action_scale, action_bias, ei_src, ei_dst):
    A = jnp.zeros((N, N), jnp.float32).at[ei_dst, ei_src].add(1.0)
    deg = jnp.zeros((N,), jnp.float32).at[ei_dst].add(1.0)
    Amean = A / jnp.clip(deg, 1.0)[:, None]

    K1, c1 = _fold_sage(Amean, Wl1, bl1, Wr1)
    K2, c2 = _fold_sage(Amean, Wl2, bl2, Wr2)
    K3, c3 = _fold_sage(Amean, Wl3, bl3, Wr3)
    W31 = K3 @ W1
    b31 = c3 @ W1 + b1

    xb = x.reshape(B, N)

    full = lambda *s: pl.BlockSpec(s, lambda i: (0,) * len(s))
    row = lambda n: pl.BlockSpec((1, n), lambda i: (0, 0))

    out = pl.pallas_call(
        _fused_net_kernel,
        grid=(B // TILE,),
        in_specs=[
            pl.BlockSpec((TILE, N), lambda i: (i, 0)),
            full(N, 6 * N), row(6 * N),
            full(6 * N, 6 * N), row(6 * N),
            full(6 * N, 128), row(128),
            full(128, 128), row(128),
            full(128, 8), row(8),
            row(8), row(8),
        ],
        out_specs=pl.BlockSpec((TILE, 8), lambda i: (i, 0)),
        out_shape=jax.ShapeDtypeStruct((B, 8), jnp.float32),
    )(xb, K1, c1.reshape(1, -1), K2, c2.reshape(1, -1), W31, b31.reshape(1, -1),
      W2, b2.reshape(1, -1), Wmu, bmu.reshape(1, -1),
      action_scale.reshape(1, -1), action_bias.reshape(1, -1))
    return out


# speedup vs baseline: 2.2451x; 2.2451x over previous
import jax
import jax.numpy as jnp
from jax.experimental import pallas as pl

B = 16384
N = 49
TILE = 512


def _fused_net_kernel(x_ref, k1_ref, c1_ref, k2_ref, c2_ref, w31_ref, b31_ref,
                      w2_ref, b2_ref, wmu_ref, cmu_ref,
                      scale_ref, bias_ref, out_ref):
    x = x_ref[...]
    h = jax.nn.relu(jnp.dot(x, k1_ref[...], preferred_element_type=jnp.float32)
                    + c1_ref[...])
    h = jax.nn.relu(jnp.dot(h, k2_ref[...], preferred_element_type=jnp.float32)
                    + c2_ref[...])
    h = jax.nn.relu(jnp.dot(h, w31_ref[...], preferred_element_type=jnp.float32)
                    + b31_ref[...])
    h = jax.nn.relu(jnp.dot(h, w2_ref[...], preferred_element_type=jnp.float32)
                    + b2_ref[...])
    mu = jnp.tanh(jnp.dot(h, wmu_ref[...], preferred_element_type=jnp.float32)
                  + cmu_ref[...])
    out_ref[...] = mu * scale_ref[...] + bias_ref[...]


def _fold_sage(Amean, Wl, bl, Wr):
    ci, co = Wl.shape
    eye = jnp.eye(N, dtype=jnp.float32)
    K = (Amean.T[:, None, :, None] * Wl[None, :, None, :]
         + eye[:, None, :, None] * Wr[None, :, None, :])
    K = K.reshape(N * ci, N * co)
    c = jnp.tile(bl, N)
    return K, c


def kernel(x, Wl1, bl1, Wr1, Wl2, bl2, Wr2, Wl3, bl3, Wr3, W1, b1, W2, b2,
           Wmu, bmu, action_scale, action_bias, ei_src, ei_dst):
    A = jnp.zeros((N, N), jnp.float32).at[ei_dst, ei_src].add(1.0)
    deg = jnp.zeros((N,), jnp.float32).at[ei_dst].add(1.0)
    Amean = A / jnp.clip(deg, 1.0)[:, None]

    K1, c1 = _fold_sage(Amean, Wl1, bl1, Wr1)
    K2, c2 = _fold_sage(Amean, Wl2, bl2, Wr2)
    K3, c3 = _fold_sage(Amean, Wl3, bl3, Wr3)
    W31 = K3 @ W1
    b31 = c3 @ W1 + b1

    xb = x.reshape(B, N)

    full = lambda *s: pl.BlockSpec(s, lambda i: (0,) * len(s))
    row = lambda n: pl.BlockSpec((1, n), lambda i: (0, 0))

    out = pl.pallas_call(
        _fused_net_kernel,
        grid=(B // TILE,),
        in_specs=[
            pl.BlockSpec((TILE, N), lambda i: (i, 0)),
            full(N, 6 * N), row(6 * N),
            full(6 * N, 6 * N), row(6 * N),
            full(6 * N, 128), row(128),
            full(128, 128), row(128),
            full(128, 8), row(8),
            row(8), row(8),
        ],
        out_specs=pl.BlockSpec((TILE, 8), lambda i: (i, 0)),
        out_shape=jax.ShapeDtypeStruct((B, 8), jnp.float32),
    )(xb, K1, c1.reshape(1, -1), K2, c2.reshape(1, -1), W31, b31.reshape(1, -1),
      W2, b2.reshape(1, -1), Wmu, bmu.reshape(1, -1),
      action_scale.reshape(1, -1), action_bias.reshape(1, -1))
    return out


# speedup vs baseline: 3.3295x; 1.4830x over previous
import jax
import jax.numpy as jnp
from jax.experimental import pallas as pl
from jax.experimental.pallas import tpu as pltpu

B = 16384
N = 49
E = 2 * N
TILE = 512


def _fused_net_kernel(x_ref, src_ref, dst_ref,
                      wl1_ref, bl1_ref, wr1_ref,
                      wl2_ref, bl2_ref, wr2_ref,
                      wl3_ref, bl3_ref, wr3_ref,
                      w1_ref, b1_ref, w2_ref, b2_ref, wmu_ref, cmu_ref,
                      scale_ref, bias_ref, out_ref,
                      k1_s, c1_s, k2_s, c2_s, w31_s, b31_s):
    f32 = jnp.float32

    @pl.when(pl.program_id(0) == 0)
    def _prep():
        dot = lambda a, b: jnp.dot(a, b, preferred_element_type=f32)
        m_sub = jax.lax.broadcasted_iota(jnp.int32, (N, E), 0)
        st = (src_ref[...] == m_sub).astype(f32)
        n_lane = jax.lax.broadcasted_iota(jnp.int32, (E, N), 1)
        d = (dst_ref[...] == n_lane).astype(f32)
        at = dot(st, d)
        deg = jnp.sum(d, axis=0, keepdims=True)
        amt = at / jnp.clip(deg, 1.0, None)

        j6 = jax.lax.broadcasted_iota(jnp.int32, (6 * N, 6), 0)
        i6 = jax.lax.broadcasted_iota(jnp.int32, (6 * N, 6), 1)
        h6 = (j6 % 6 == i6).astype(f32)
        e6 = jax.lax.broadcasted_iota(jnp.int32, (6 * N, N), 0)
        n6 = jax.lax.broadcasted_iota(jnp.int32, (6 * N, N), 1)
        eb6 = (e6 // 6 == n6).astype(f32)
        k6 = jax.lax.broadcasted_iota(jnp.int32, (6, 6 * N), 1)
        o6 = jax.lax.broadcasted_iota(jnp.int32, (6, 6 * N), 0)
        h6b = (k6 % 6 == o6).astype(f32)
        f6 = (jax.lax.broadcasted_iota(jnp.int32, (N, 6 * N), 1) // 6
              == jax.lax.broadcasted_iota(jnp.int32, (N, 6 * N), 0)).astype(f32)
        k12 = jax.lax.broadcasted_iota(jnp.int32, (12, 12 * N), 1)
        o12 = jax.lax.broadcasted_iota(jnp.int32, (12, 12 * N), 0)
        h12b = (k12 % 12 == o12).astype(f32)
        f12 = (jax.lax.broadcasted_iota(jnp.int32, (N, 12 * N), 1) // 12
               == jax.lax.broadcasted_iota(jnp.int32, (N, 12 * N), 0)).astype(f32)

        r66 = jax.lax.broadcasted_iota(jnp.int32, (6 * N, 6 * N), 0)
        cc66 = jax.lax.broadcasted_iota(jnp.int32, (6 * N, 6 * N), 1)
        blk66 = (r66 // 6 == cc66 // 6).astype(f32)
        r612 = jax.lax.broadcasted_iota(jnp.int32, (6 * N, 12 * N), 0)
        c612 = jax.lax.broadcasted_iota(jnp.int32, (6 * N, 12 * N), 1)
        blk612 = (r612 // 6 == c612 // 12).astype(f32)

        amt_f6 = dot(amt, f6)
        wl1_row = dot(wl1_ref[...], h6b)
        wr1_row = dot(wr1_ref[...], h6b)
        eye_f6 = (jax.lax.broadcasted_iota(jnp.int32, (N, 6 * N), 1) // 6
                  == jax.lax.broadcasted_iota(jnp.int32, (N, 6 * N), 0)).astype(f32)
        k1_s[...] = amt_f6 * wl1_row + eye_f6 * wr1_row
        c1_s[...] = dot(bl1_ref[...], h6b)

        aexp2 = dot(dot(eb6, amt), f6)
        tile_wl2 = dot(dot(h6, wl2_ref[...]), h6b)
        tile_wr2 = dot(dot(h6, wr2_ref[...]), h6b)
        k2_s[...] = aexp2 * tile_wl2 + blk66 * tile_wr2
        c2_s[...] = dot(bl2_ref[...], h6b)

        aexp3 = dot(dot(eb6, amt), f12)
        tile_wl3 = dot(dot(h6, wl3_ref[...]), h12b)
        tile_wr3 = dot(dot(h6, wr3_ref[...]), h12b)
        k3 = aexp3 * tile_wl3 + blk612 * tile_wr3
        w31_s[...] = dot(k3, w1_ref[...])
        b31_s[...] = dot(dot(bl3_ref[...], h12b), w1_ref[...]) + b1_ref[...]

    x = x_ref[...]
    h = jax.nn.relu(jnp.dot(x, k1_s[...], preferred_element_type=f32)
                    + c1_s[...])
    h = jax.nn.relu(jnp.dot(h, k2_s[...], preferred_element_type=f32)
                    + c2_s[...])
    h = jax.nn.relu(jnp.dot(h, w31_s[...], preferred_element_type=f32)
                    + b31_s[...])
    h = jax.nn.relu(jnp.dot(h, w2_ref[...], preferred_element_type=f32)
                    + b2_ref[...])
    mu = jnp.tanh(jnp.dot(h, wmu_ref[...], preferred_element_type=f32)
                  + cmu_ref[...])
    out_ref[...] = mu * scale_ref[...] + bias_ref[...]


def kernel(x, Wl1, bl1, Wr1, Wl2, bl2, Wr2, Wl3, bl3, Wr3, W1, b1, W2, b2,
           Wmu, bmu, action_scale, action_bias, ei_src, ei_dst):
    xb = x.reshape(B, N)

    full = lambda *s: pl.BlockSpec(s, lambda i: (0,) * len(s))
    row = lambda n: pl.BlockSpec((1, n), lambda i: (0, 0))

    out = pl.pallas_call(
        _fused_net_kernel,
        grid=(B // TILE,),
        in_specs=[
            pl.BlockSpec((TILE, N), lambda i: (i, 0)),
            row(E), full(E, 1),
            row(6), row(6), row(6),
            full(6, 6), row(6), full(6, 6),
            full(6, 12), row(12), full(6, 12),
            full(12 * N, 128), row(128),
            full(128, 128), row(128),
            full(128, 8), row(8),
            row(8), row(8),
        ],
        out_specs=pl.BlockSpec((TILE, 8), lambda i: (i, 0)),
        out_shape=jax.ShapeDtypeStruct((B, 8), jnp.float32),
        scratch_shapes=[
            pltpu.VMEM((N, 6 * N), jnp.float32),
            pltpu.VMEM((1, 6 * N), jnp.float32),
            pltpu.VMEM((6 * N, 6 * N), jnp.float32),
            pltpu.VMEM((1, 6 * N), jnp.float32),
            pltpu.VMEM((6 * N, 128), jnp.float32),
            pltpu.VMEM((1, 128), jnp.float32),
        ],
    )(xb, ei_src.reshape(1, E), ei_dst.reshape(E, 1),
      Wl1.reshape(1, 6), bl1.reshape(1, 6), Wr1.reshape(1, 6),
      Wl2, bl2.reshape(1, 6), Wr2,
      Wl3, bl3.reshape(1, 12), Wr3,
      W1, b1.reshape(1, -1), W2, b2.reshape(1, -1), Wmu, bmu.reshape(1, -1),
      action_scale.reshape(1, -1), action_bias.reshape(1, -1))
    return out


# speedup vs baseline: 4.0551x; 1.2179x over previous
import jax
import jax.numpy as jnp
from jax.experimental import pallas as pl
from jax.experimental.pallas import tpu as pltpu

B = 16384
N = 49
E = 2 * N
TILE = 1024


def _fused_net_kernel(x_ref, src_ref, dst_ref,
                      wl1_ref, bl1_ref, wr1_ref,
                      wl2_ref, bl2_ref, wr2_ref,
                      wl3_ref, bl3_ref, wr3_ref,
                      w1_ref, b1_ref, w2_ref, b2_ref, wmu_ref, cmu_ref,
                      scale_ref, bias_ref, out_ref,
                      k1_s, c1_s, k2_s, c2_s, w31_s, b31_s):
    f32 = jnp.float32

    @pl.when(pl.program_id(0) == 0)
    def _prep():
        dot = lambda a, b: jnp.dot(a, b, preferred_element_type=f32)
        m_sub = jax.lax.broadcasted_iota(jnp.int32, (N, E), 0)
        st = (src_ref[...] == m_sub).astype(f32)
        n_lane = jax.lax.broadcasted_iota(jnp.int32, (E, N), 1)
        d = (dst_ref[...] == n_lane).astype(f32)
        at = dot(st, d)
        deg = jnp.sum(d, axis=0, keepdims=True)
        amt = at / jnp.clip(deg, 1.0, None)

        j6 = jax.lax.broadcasted_iota(jnp.int32, (6 * N, 6), 0)
        i6 = jax.lax.broadcasted_iota(jnp.int32, (6 * N, 6), 1)
        h6 = (j6 % 6 == i6).astype(f32)
        e6 = jax.lax.broadcasted_iota(jnp.int32, (6 * N, N), 0)
        n6 = jax.lax.broadcasted_iota(jnp.int32, (6 * N, N), 1)
        eb6 = (e6 // 6 == n6).astype(f32)
        k6 = jax.lax.broadcasted_iota(jnp.int32, (6, 6 * N), 1)
        o6 = jax.lax.broadcasted_iota(jnp.int32, (6, 6 * N), 0)
        h6b = (k6 % 6 == o6).astype(f32)
        f6 = (jax.lax.broadcasted_iota(jnp.int32, (N, 6 * N), 1) // 6
              == jax.lax.broadcasted_iota(jnp.int32, (N, 6 * N), 0)).astype(f32)
        k12 = jax.lax.broadcasted_iota(jnp.int32, (12, 12 * N), 1)
        o12 = jax.lax.broadcasted_iota(jnp.int32, (12, 12 * N), 0)
        h12b = (k12 % 12 == o12).astype(f32)
        f12 = (jax.lax.broadcasted_iota(jnp.int32, (N, 12 * N), 1) // 12
               == jax.lax.broadcasted_iota(jnp.int32, (N, 12 * N), 0)).astype(f32)

        r66 = jax.lax.broadcasted_iota(jnp.int32, (6 * N, 6 * N), 0)
        cc66 = jax.lax.broadcasted_iota(jnp.int32, (6 * N, 6 * N), 1)
        blk66 = (r66 // 6 == cc66 // 6).astype(f32)
        r612 = jax.lax.broadcasted_iota(jnp.int32, (6 * N, 12 * N), 0)
        c612 = jax.lax.broadcasted_iota(jnp.int32, (6 * N, 12 * N), 1)
        blk612 = (r612 // 6 == c612 // 12).astype(f32)

        amt_f6 = dot(amt, f6)
        wl1_row = dot(wl1_ref[...], h6b)
        wr1_row = dot(wr1_ref[...], h6b)
        eye_f6 = (jax.lax.broadcasted_iota(jnp.int32, (N, 6 * N), 1) // 6
                  == jax.lax.broadcasted_iota(jnp.int32, (N, 6 * N), 0)).astype(f32)
        k1_s[...] = amt_f6 * wl1_row + eye_f6 * wr1_row
        c1_s[...] = dot(bl1_ref[...], h6b)

        aexp2 = dot(dot(eb6, amt), f6)
        tile_wl2 = dot(dot(h6, wl2_ref[...]), h6b)
        tile_wr2 = dot(dot(h6, wr2_ref[...]), h6b)
        k2_s[...] = aexp2 * tile_wl2 + blk66 * tile_wr2
        c2_s[...] = dot(bl2_ref[...], h6b)

        aexp3 = dot(dot(eb6, amt), f12)
        tile_wl3 = dot(dot(h6, wl3_ref[...]), h12b)
        tile_wr3 = dot(dot(h6, wr3_ref[...]), h12b)
        k3 = aexp3 * tile_wl3 + blk612 * tile_wr3
        w31_s[...] = dot(k3, w1_ref[...])
        b31_s[...] = dot(dot(bl3_ref[...], h12b), w1_ref[...]) + b1_ref[...]

    x = x_ref[...]
    h = jax.nn.relu(jnp.dot(x, k1_s[...], preferred_element_type=f32)
                    + c1_s[...])
    h = jax.nn.relu(jnp.dot(h, k2_s[...], preferred_element_type=f32)
                    + c2_s[...])
    h = jax.nn.relu(jnp.dot(h, w31_s[...], preferred_element_type=f32)
                    + b31_s[...])
    h = jax.nn.relu(jnp.dot(h, w2_ref[...], preferred_element_type=f32)
                    + b2_ref[...])
    mu = jnp.tanh(jnp.dot(h, wmu_ref[...], preferred_element_type=f32)
                  + cmu_ref[...])
    out_ref[...] = mu * scale_ref[...] + bias_ref[...]


def kernel(x, Wl1, bl1, Wr1, Wl2, bl2, Wr2, Wl3, bl3, Wr3, W1, b1, W2, b2,
           Wmu, bmu, action_scale, action_bias, ei_src, ei_dst):
    xb = x.reshape(B, N)

    full = lambda *s: pl.BlockSpec(s, lambda i: (0,) * len(s))
    row = lambda n: pl.BlockSpec((1, n), lambda i: (0, 0))

    out = pl.pallas_call(
        _fused_net_kernel,
        grid=(B // TILE,),
        in_specs=[
            pl.BlockSpec((TILE, N), lambda i: (i, 0)),
            row(E), full(E, 1),
            row(6), row(6), row(6),
            full(6, 6), row(6), full(6, 6),
            full(6, 12), row(12), full(6, 12),
            full(12 * N, 128), row(128),
            full(128, 128), row(128),
            full(128, 8), row(8),
            row(8), row(8),
        ],
        out_specs=pl.BlockSpec((TILE, 8), lambda i: (i, 0)),
        out_shape=jax.ShapeDtypeStruct((B, 8), jnp.float32),
        scratch_shapes=[
            pltpu.VMEM((N, 6 * N), jnp.float32),
            pltpu.VMEM((1, 6 * N), jnp.float32),
            pltpu.VMEM((6 * N, 6 * N), jnp.float32),
            pltpu.VMEM((1, 6 * N), jnp.float32),
            pltpu.VMEM((6 * N, 128), jnp.float32),
            pltpu.VMEM((1, 128), jnp.float32),
        ],
    )(xb, ei_src.reshape(1, E), ei_dst.reshape(E, 1),
      Wl1.reshape(1, 6), bl1.reshape(1, 6), Wr1.reshape(1, 6),
      Wl2, bl2.reshape(1, 6), Wr2,
      Wl3, bl3.reshape(1, 12), Wr3,
      W1, b1.reshape(1, -1), W2, b2.reshape(1, -1), Wmu, bmu.reshape(1, -1),
      action_scale.reshape(1, -1), action_bias.reshape(1, -1))
    return out


# speedup vs baseline: 4.2985x; 1.0600x over previous
import jax
import jax.numpy as jnp
from jax.experimental import pallas as pl
from jax.experimental.pallas import tpu as pltpu

B = 16384
N = 49
E = 2 * N
TILE = 2048


def _fused_net_kernel(x_ref, src_ref, dst_ref,
                      wl1_ref, bl1_ref, wr1_ref,
                      wl2_ref, bl2_ref, wr2_ref,
                      wl3_ref, bl3_ref, wr3_ref,
                      w1_ref, b1_ref, w2_ref, b2_ref, wmu_ref, cmu_ref,
                      scale_ref, bias_ref, out_ref,
                      k1_s, c1_s, k2_s, c2_s, w31_s, b31_s):
    f32 = jnp.float32

    @pl.when(pl.program_id(0) == 0)
    def _prep():
        dot = lambda a, b: jnp.dot(a, b, preferred_element_type=f32)
        m_sub = jax.lax.broadcasted_iota(jnp.int32, (N, E), 0)
        st = (src_ref[...] == m_sub).astype(f32)
        n_lane = jax.lax.broadcasted_iota(jnp.int32, (E, N), 1)
        d = (dst_ref[...] == n_lane).astype(f32)
        at = dot(st, d)
        deg = jnp.sum(d, axis=0, keepdims=True)
        amt = at / jnp.clip(deg, 1.0, None)

        j6 = jax.lax.broadcasted_iota(jnp.int32, (6 * N, 6), 0)
        i6 = jax.lax.broadcasted_iota(jnp.int32, (6 * N, 6), 1)
        h6 = (j6 % 6 == i6).astype(f32)
        e6 = jax.lax.broadcasted_iota(jnp.int32, (6 * N, N), 0)
        n6 = jax.lax.broadcasted_iota(jnp.int32, (6 * N, N), 1)
        eb6 = (e6 // 6 == n6).astype(f32)
        k6 = jax.lax.broadcasted_iota(jnp.int32, (6, 6 * N), 1)
        o6 = jax.lax.broadcasted_iota(jnp.int32, (6, 6 * N), 0)
        h6b = (k6 % 6 == o6).astype(f32)
        f6 = (jax.lax.broadcasted_iota(jnp.int32, (N, 6 * N), 1) // 6
              == jax.lax.broadcasted_iota(jnp.int32, (N, 6 * N), 0)).astype(f32)
        k12 = jax.lax.broadcasted_iota(jnp.int32, (12, 12 * N), 1)
        o12 = jax.lax.broadcasted_iota(jnp.int32, (12, 12 * N), 0)
        h12b = (k12 % 12 == o12).astype(f32)
        f12 = (jax.lax.broadcasted_iota(jnp.int32, (N, 12 * N), 1) // 12
               == jax.lax.broadcasted_iota(jnp.int32, (N, 12 * N), 0)).astype(f32)

        r66 = jax.lax.broadcasted_iota(jnp.int32, (6 * N, 6 * N), 0)
        cc66 = jax.lax.broadcasted_iota(jnp.int32, (6 * N, 6 * N), 1)
        blk66 = (r66 // 6 == cc66 // 6).astype(f32)
        r612 = jax.lax.broadcasted_iota(jnp.int32, (6 * N, 12 * N), 0)
        c612 = jax.lax.broadcasted_iota(jnp.int32, (6 * N, 12 * N), 1)
        blk612 = (r612 // 6 == c612 // 12).astype(f32)

        amt_f6 = dot(amt, f6)
        wl1_row = dot(wl1_ref[...], h6b)
        wr1_row = dot(wr1_ref[...], h6b)
        eye_f6 = (jax.lax.broadcasted_iota(jnp.int32, (N, 6 * N), 1) // 6
                  == jax.lax.broadcasted_iota(jnp.int32, (N, 6 * N), 0)).astype(f32)
        k1_s[...] = amt_f6 * wl1_row + eye_f6 * wr1_row
        c1_s[...] = dot(bl1_ref[...], h6b)

        aexp2 = dot(dot(eb6, amt), f6)
        tile_wl2 = dot(dot(h6, wl2_ref[...]), h6b)
        tile_wr2 = dot(dot(h6, wr2_ref[...]), h6b)
        k2_s[...] = aexp2 * tile_wl2 + blk66 * tile_wr2
        c2_s[...] = dot(bl2_ref[...], h6b)

        aexp3 = dot(dot(eb6, amt), f12)
        tile_wl3 = dot(dot(h6, wl3_ref[...]), h12b)
        tile_wr3 = dot(dot(h6, wr3_ref[...]), h12b)
        k3 = aexp3 * tile_wl3 + blk612 * tile_wr3
        w31_s[...] = dot(k3, w1_ref[...])
        b31_s[...] = dot(dot(bl3_ref[...], h12b), w1_ref[...]) + b1_ref[...]

    x = x_ref[...]
    h = jax.nn.relu(jnp.dot(x, k1_s[...], preferred_element_type=f32)
                    + c1_s[...])
    h = jax.nn.relu(jnp.dot(h, k2_s[...], preferred_element_type=f32)
                    + c2_s[...])
    h = jax.nn.relu(jnp.dot(h, w31_s[...], preferred_element_type=f32)
                    + b31_s[...])
    h = jax.nn.relu(jnp.dot(h, w2_ref[...], preferred_element_type=f32)
                    + b2_ref[...])
    mu = jnp.tanh(jnp.dot(h, wmu_ref[...], preferred_element_type=f32)
                  + cmu_ref[...])
    out_ref[...] = mu * scale_ref[...] + bias_ref[...]


def kernel(x, Wl1, bl1, Wr1, Wl2, bl2, Wr2, Wl3, bl3, Wr3, W1, b1, W2, b2,
           Wmu, bmu, action_scale, action_bias, ei_src, ei_dst):
    xb = x.reshape(B, N)

    full = lambda *s: pl.BlockSpec(s, lambda i: (0,) * len(s))
    row = lambda n: pl.BlockSpec((1, n), lambda i: (0, 0))

    out = pl.pallas_call(
        _fused_net_kernel,
        grid=(B // TILE,),
        in_specs=[
            pl.BlockSpec((TILE, N), lambda i: (i, 0)),
            row(E), full(E, 1),
            row(6), row(6), row(6),
            full(6, 6), row(6), full(6, 6),
            full(6, 12), row(12), full(6, 12),
            full(12 * N, 128), row(128),
            full(128, 128), row(128),
            full(128, 8), row(8),
            row(8), row(8),
        ],
        out_specs=pl.BlockSpec((TILE, 8), lambda i: (i, 0)),
        out_shape=jax.ShapeDtypeStruct((B, 8), jnp.float32),
        scratch_shapes=[
            pltpu.VMEM((N, 6 * N), jnp.float32),
            pltpu.VMEM((1, 6 * N), jnp.float32),
            pltpu.VMEM((6 * N, 6 * N), jnp.float32),
            pltpu.VMEM((1, 6 * N), jnp.float32),
            pltpu.VMEM((6 * N, 128), jnp.float32),
            pltpu.VMEM((1, 128), jnp.float32),
        ],
    )(xb, ei_src.reshape(1, E), ei_dst.reshape(E, 1),
      Wl1.reshape(1, 6), bl1.reshape(1, 6), Wr1.reshape(1, 6),
      Wl2, bl2.reshape(1, 6), Wr2,
      Wl3, bl3.reshape(1, 12), Wr3,
      W1, b1.reshape(1, -1), W2, b2.reshape(1, -1), Wmu, bmu.reshape(1, -1),
      action_scale.reshape(1, -1), action_bias.reshape(1, -1))
    return out


# speedup vs baseline: 4.3611x; 1.0146x over previous
import jax
import jax.numpy as jnp
from jax.experimental import pallas as pl
from jax.experimental.pallas import tpu as pltpu

B = 16384
N = 49
E = 2 * N
TILE = 4096


def _fused_net_kernel(x_ref, src_ref, dst_ref,
                      wl1_ref, bl1_ref, wr1_ref,
                      wl2_ref, bl2_ref, wr2_ref,
                      wl3_ref, bl3_ref, wr3_ref,
                      w1_ref, b1_ref, w2_ref, b2_ref, wmu_ref, cmu_ref,
                      scale_ref, bias_ref, out_ref,
                      k1_s, c1_s, k2_s, c2_s, w31_s, b31_s):
    f32 = jnp.float32

    @pl.when(pl.program_id(0) == 0)
    def _prep():
        dot = lambda a, b: jnp.dot(a, b, preferred_element_type=f32)
        m_sub = jax.lax.broadcasted_iota(jnp.int32, (N, E), 0)
        st = (src_ref[...] == m_sub).astype(f32)
        n_lane = jax.lax.broadcasted_iota(jnp.int32, (E, N), 1)
        d = (dst_ref[...] == n_lane).astype(f32)
        at = dot(st, d)
        deg = jnp.sum(d, axis=0, keepdims=True)
        amt = at / jnp.clip(deg, 1.0, None)

        j6 = jax.lax.broadcasted_iota(jnp.int32, (6 * N, 6), 0)
        i6 = jax.lax.broadcasted_iota(jnp.int32, (6 * N, 6), 1)
        h6 = (j6 % 6 == i6).astype(f32)
        e6 = jax.lax.broadcasted_iota(jnp.int32, (6 * N, N), 0)
        n6 = jax.lax.broadcasted_iota(jnp.int32, (6 * N, N), 1)
        eb6 = (e6 // 6 == n6).astype(f32)
        k6 = jax.lax.broadcasted_iota(jnp.int32, (6, 6 * N), 1)
        o6 = jax.lax.broadcasted_iota(jnp.int32, (6, 6 * N), 0)
        h6b = (k6 % 6 == o6).astype(f32)
        f6 = (jax.lax.broadcasted_iota(jnp.int32, (N, 6 * N), 1) // 6
              == jax.lax.broadcasted_iota(jnp.int32, (N, 6 * N), 0)).astype(f32)
        k12 = jax.lax.broadcasted_iota(jnp.int32, (12, 12 * N), 1)
        o12 = jax.lax.broadcasted_iota(jnp.int32, (12, 12 * N), 0)
        h12b = (k12 % 12 == o12).astype(f32)
        f12 = (jax.lax.broadcasted_iota(jnp.int32, (N, 12 * N), 1) // 12
               == jax.lax.broadcasted_iota(jnp.int32, (N, 12 * N), 0)).astype(f32)

        r66 = jax.lax.broadcasted_iota(jnp.int32, (6 * N, 6 * N), 0)
        cc66 = jax.lax.broadcasted_iota(jnp.int32, (6 * N, 6 * N), 1)
        blk66 = (r66 // 6 == cc66 // 6).astype(f32)
        r612 = jax.lax.broadcasted_iota(jnp.int32, (6 * N, 12 * N), 0)
        c612 = jax.lax.broadcasted_iota(jnp.int32, (6 * N, 12 * N), 1)
        blk612 = (r612 // 6 == c612 // 12).astype(f32)

        amt_f6 = dot(amt, f6)
        wl1_row = dot(wl1_ref[...], h6b)
        wr1_row = dot(wr1_ref[...], h6b)
        eye_f6 = (jax.lax.broadcasted_iota(jnp.int32, (N, 6 * N), 1) // 6
                  == jax.lax.broadcasted_iota(jnp.int32, (N, 6 * N), 0)).astype(f32)
        k1_s[...] = amt_f6 * wl1_row + eye_f6 * wr1_row
        c1_s[...] = dot(bl1_ref[...], h6b)

        aexp2 = dot(dot(eb6, amt), f6)
        tile_wl2 = dot(dot(h6, wl2_ref[...]), h6b)
        tile_wr2 = dot(dot(h6, wr2_ref[...]), h6b)
        k2_s[...] = aexp2 * tile_wl2 + blk66 * tile_wr2
        c2_s[...] = dot(bl2_ref[...], h6b)

        aexp3 = dot(dot(eb6, amt), f12)
        tile_wl3 = dot(dot(h6, wl3_ref[...]), h12b)
        tile_wr3 = dot(dot(h6, wr3_ref[...]), h12b)
        k3 = aexp3 * tile_wl3 + blk612 * tile_wr3
        w31_s[...] = dot(k3, w1_ref[...])
        b31_s[...] = dot(dot(bl3_ref[...], h12b), w1_ref[...]) + b1_ref[...]

    x = x_ref[...]
    h = jax.nn.relu(jnp.dot(x, k1_s[...], preferred_element_type=f32)
                    + c1_s[...])
    h = jax.nn.relu(jnp.dot(h, k2_s[...], preferred_element_type=f32)
                    + c2_s[...])
    h = jax.nn.relu(jnp.dot(h, w31_s[...], preferred_element_type=f32)
                    + b31_s[...])
    h = jax.nn.relu(jnp.dot(h, w2_ref[...], preferred_element_type=f32)
                    + b2_ref[...])
    mu = jnp.tanh(jnp.dot(h, wmu_ref[...], preferred_element_type=f32)
                  + cmu_ref[...])
    out_ref[...] = mu * scale_ref[...] + bias_ref[...]


def kernel(x, Wl1, bl1, Wr1, Wl2, bl2, Wr2, Wl3, bl3, Wr3, W1, b1, W2, b2,
           Wmu, bmu, action_scale, action_bias, ei_src, ei_dst):
    xb = x.reshape(B, N)

    full = lambda *s: pl.BlockSpec(s, lambda i: (0,) * len(s))
    row = lambda n: pl.BlockSpec((1, n), lambda i: (0, 0))

    out = pl.pallas_call(
        _fused_net_kernel,
        grid=(B // TILE,),
        in_specs=[
            pl.BlockSpec((TILE, N), lambda i: (i, 0)),
            row(E), full(E, 1),
            row(6), row(6), row(6),
            full(6, 6), row(6), full(6, 6),
            full(6, 12), row(12), full(6, 12),
            full(12 * N, 128), row(128),
            full(128, 128), row(128),
            full(128, 8), row(8),
            row(8), row(8),
        ],
        out_specs=pl.BlockSpec((TILE, 8), lambda i: (i, 0)),
        out_shape=jax.ShapeDtypeStruct((B, 8), jnp.float32),
        scratch_shapes=[
            pltpu.VMEM((N, 6 * N), jnp.float32),
            pltpu.VMEM((1, 6 * N), jnp.float32),
            pltpu.VMEM((6 * N, 6 * N), jnp.float32),
            pltpu.VMEM((1, 6 * N), jnp.float32),
            pltpu.VMEM((6 * N, 128), jnp.float32),
            pltpu.VMEM((1, 128), jnp.float32),
        ],
    )(xb, ei_src.reshape(1, E), ei_dst.reshape(E, 1),
      Wl1.reshape(1, 6), bl1.reshape(1, 6), Wr1.reshape(1, 6),
      Wl2, bl2.reshape(1, 6), Wr2,
      Wl3, bl3.reshape(1, 12), Wr3,
      W1, b1.reshape(1, -1), W2, b2.reshape(1, -1), Wmu, bmu.reshape(1, -1),
      action_scale.reshape(1, -1), action_bias.reshape(1, -1))
    return out
